# Initial kernel scaffold; baseline (speedup 1.0000x reference)
import jax
import jax.numpy as jnp
from jax.experimental import pallas as pl


def kernel(nodes, num_atoms, node_index, state, max_num_nbrs, num_pairs, edge_index, index, edges, params):
    raise NotImplementedError("write your pallas kernel here")


# speedup vs baseline: 3.2366x; 3.2366x over previous
import functools

import jax
import jax.numpy as jnp
import numpy as np
from jax.experimental import pallas as pl
from jax.experimental.pallas import tpu as pltpu
from jax.experimental.pallas import tpu_sc as plsc

_LOG2 = float(np.log(2.0))

_SC_CORES = 2
_SC_SUBCORES = 16


def _ssp(x):
    return jnp.maximum(x, 0.0) + jnp.log1p(jnp.exp(-jnp.abs(x))) - _LOG2


def _sc_gather(table, idx):
    K = idx.shape[0]
    D = table.shape[1]
    W = 128
    mesh = plsc.VectorSubcoreMesh(core_axis_name="core", subcore_axis_name="subcore")

    @pl.kernel(out_type=jax.ShapeDtypeStruct((K, D), table.dtype), mesh=mesh)
    def k(t_hbm, i_hbm, o_hbm):
        def body(i_vmem, o_vmem):
            pltpu.sync_copy(t_hbm.at[i_vmem.at[0]], o_vmem)

        pltpu.emit_pipeline(
            body,
            grid=(K // W,),
            in_specs=[pl.BlockSpec((1, W), lambda i: (0, i))],
            out_specs=[pl.BlockSpec((W, D), lambda i: (i, 0))],
            core_axis_name=("core", "subcore"),
            dimension_semantics=(pltpu.PARALLEL,),
        )(i_hbm, o_hbm)

    return k(table, idx.reshape(1, K))


def _sc_scatter_add(ep, idx, n_rows):
    E, H = ep.shape
    CHUNK = 128
    n_pad = ((n_rows + 8 * _SC_SUBCORES - 1) // (8 * _SC_SUBCORES)) * (8 * _SC_SUBCORES)
    RPT = n_pad // _SC_SUBCORES
    mesh = plsc.VectorSubcoreMesh(core_axis_name="core", subcore_axis_name="subcore")
    zeros = jnp.zeros((RPT, H), dtype=ep.dtype)

    @pl.kernel(
        out_type=jax.ShapeDtypeStruct((_SC_CORES, n_pad, H), ep.dtype),
        mesh=mesh,
        scratch_types=[pltpu.VMEM_SHARED((n_pad, H), ep.dtype)],
    )
    def k(ep_hbm, i_hbm, z_hbm, o_hbm, acc):
        core = jax.lax.axis_index("core")
        sub = jax.lax.axis_index("subcore")
        pltpu.sync_copy(z_hbm, acc.at[pl.ds(sub * RPT, RPT)])
        plsc.subcore_barrier()

        def body(ep_vmem, i_vmem):
            pltpu.sync_copy(ep_vmem, acc.at[i_vmem.at[0]], add=True)

        pltpu.emit_pipeline(
            body,
            grid=(E // CHUNK,),
            in_specs=[
                pl.BlockSpec((CHUNK, H), lambda i: (i, 0)),
                pl.BlockSpec((1, CHUNK), lambda i: (0, i)),
            ],
            out_specs=[],
            core_axis_name=("core", "subcore"),
            dimension_semantics=(pltpu.PARALLEL,),
        )(ep_hbm, i_hbm)
        plsc.subcore_barrier()
        pltpu.sync_copy(
            acc.at[pl.ds(sub * RPT, RPT)],
            o_hbm.at[core].at[pl.ds(sub * RPT, RPT)],
        )

    return k(ep, idx.reshape(1, E), zeros)


def _k0_body(state_ref, wu1_ref, bu1_ref, wu2_ref, bu2_ref, wfeu_ref, wfvu_ref,
             u_ref, gu1_ref, gv1_ref):
    t = _ssp(state_ref[...] @ wu1_ref[...] + bu1_ref[...])
    u = _ssp(t @ wu2_ref[...] + bu2_ref[...])
    u_ref[...] = u
    gu1_ref[...] = u @ wfeu_ref[...]
    gv1_ref[...] = u @ wfvu_ref[...]


def _k1_body(nodes_ref, wv1_ref, bv1_ref, wv2_ref, bv2_ref, wfec_ref, wfen_ref,
             v_ref, vc_ref, vn_ref):
    t = _ssp(nodes_ref[...] @ wv1_ref[...] + bv1_ref[...])
    v = _ssp(t @ wv2_ref[...] + bv2_ref[...])
    v_ref[...] = v
    vc_ref[...] = v @ wfec_ref[...]
    vn_ref[...] = v @ wfen_ref[...]


def _k2_body(edges_ref, we1_ref, be1_ref, we2_ref, be2_ref, wfee_ref,
             epre_ref):
    t = _ssp(edges_ref[...] @ we1_ref[...] + be1_ref[...])
    e = _ssp(t @ we2_ref[...] + be2_ref[...])
    epre_ref[...] = e @ wfee_ref[...]


def _k3_body(epre_ref, gc_ref, gn_ref, edges_ref, gu1_ref, eidx_ref,
             bfe1_ref, wfe2_ref, bfe2_ref, wfe3_ref, bfe3_ref,
             eout_ref, ep_ref, acc_ref):
    g = pl.program_id(0)
    B = epre_ref.shape[0]
    h1 = _ssp(epre_ref[...] + gc_ref[...] + gn_ref[...] + gu1_ref[0]
              + bfe1_ref[...])
    h2 = _ssp(h1 @ wfe2_ref[...] + bfe2_ref[...])
    ep = _ssp(h2 @ wfe3_ref[...] + bfe3_ref[...])
    ep_ref[...] = ep
    eout_ref[...] = edges_ref[...] + ep
    eidx = eidx_ref[0]
    oh = (jax.lax.broadcasted_iota(jnp.int32, (128, B), 0) == eidx
          ).astype(jnp.float32)
    contrib = jax.lax.dot(oh, ep)

    @pl.when(g == 0)
    def _():
        acc_ref[...] = contrib

    @pl.when(g > 0)
    def _():
        acc_ref[...] += contrib


def _k4_body(p0_ref, p1_ref, v_ref, nodes_ref, gv1_ref, nidx_ref,
             wfva_ref, wfvb_ref, bfv1_ref, wfv2_ref, bfv2_ref, wfv3_ref,
             bfv3_ref, vout_ref, acc_ref, *, npg, gpb):
    g = pl.program_id(0)
    B = v_ref.shape[0]
    row_graph = g * gpb + jax.lax.broadcasted_iota(jnp.int32, (B, 128), 0) // npg
    ohu = (row_graph == jax.lax.broadcasted_iota(jnp.int32, (B, 128), 1)
           ).astype(jnp.float32)
    uv = jax.lax.dot(ohu, gv1_ref[...])
    epav = p0_ref[0] + p1_ref[0]
    h1 = _ssp(jax.lax.dot(epav, wfva_ref[...]) + jax.lax.dot(v_ref[...], wfvb_ref[...])
              + uv + bfv1_ref[...])
    h2 = _ssp(h1 @ wfv2_ref[...] + bfv2_ref[...])
    vp = _ssp(h2 @ wfv3_ref[...] + bfv3_ref[...])
    vout_ref[...] = nodes_ref[...] + vp
    nidx = nidx_ref[0]
    oh = (jax.lax.broadcasted_iota(jnp.int32, (128, B), 0) == nidx
          ).astype(jnp.float32)
    contrib = jax.lax.dot(oh, vp)

    @pl.when(g == 0)
    def _():
        acc_ref[...] = contrib

    @pl.when(g > 0)
    def _():
        acc_ref[...] += contrib


def _k5_body(epau_ref, vpau_ref, u_ref, state_ref,
             wfua_ref, wfub_ref, wfuc_ref, bfu1_ref, wfu2_ref, bfu2_ref,
             wfu3_ref, bfu3_ref, uout_ref):
    h1 = _ssp(jax.lax.dot(epau_ref[...], wfua_ref[...])
              + jax.lax.dot(vpau_ref[...], wfub_ref[...])
              + jax.lax.dot(u_ref[...], wfuc_ref[...]) + bfu1_ref[...])
    h2 = _ssp(h1 @ wfu2_ref[...] + bfu2_ref[...])
    up = _ssp(h2 @ wfu3_ref[...] + bfu3_ref[...])
    uout_ref[...] = state_ref[...] + up


def _row2(b):
    return b.reshape(1, -1)


def kernel(nodes, num_atoms, node_index, state, max_num_nbrs, num_pairs,
           edge_index, index, edges, params):
    N, H = nodes.shape
    E = edges.shape[0]
    G = state.shape[0]
    NPG = N // G
    PPG = E // G
    H2 = 2 * H

    (wv1, bv1), (wv2, bv2) = params["pv"]
    (we1, be1), (we2, be2) = params["pe"]
    (wu1, bu1), (wu2, bu2) = params["pu"]
    (wfe1, bfe1), (wfe2, bfe2), (wfe3, bfe3) = params["fe"]
    (wfv1, bfv1), (wfv2, bfv2), (wfv3, bfv3) = params["fv"]
    (wfu1, bfu1), (wfu2, bfu2), (wfu3, bfu3) = params["fu"]

    wfe_c, wfe_e, wfe_n, wfe_u = (wfe1[0:H], wfe1[H:2 * H], wfe1[2 * H:3 * H],
                                  wfe1[3 * H:4 * H])
    wfv_a, wfv_b, wfv_u = wfv1[0:H], wfv1[H:2 * H], wfv1[2 * H:3 * H]
    wfu_a, wfu_b, wfu_c = wfu1[0:H], wfu1[H:2 * H], wfu1[2 * H:3 * H]

    GP = 128
    state_p = jnp.zeros((GP, H), jnp.float32).at[:G].set(state)

    u_p, gu1, gv1 = pl.pallas_call(
        _k0_body,
        out_shape=[
            jax.ShapeDtypeStruct((GP, H), jnp.float32),
            jax.ShapeDtypeStruct((GP, H2), jnp.float32),
            jax.ShapeDtypeStruct((GP, H2), jnp.float32),
        ],
    )(state_p, wu1, _row2(bu1), wu2, _row2(bu2), wfe_u, wfv_u)

    BN = 1000
    v, vc, vn = pl.pallas_call(
        _k1_body,
        grid=(N // BN,),
        in_specs=[
            pl.BlockSpec((BN, H), lambda i: (i, 0)),
            pl.BlockSpec((H, H2), lambda i: (0, 0)),
            pl.BlockSpec((1, H2), lambda i: (0, 0)),
            pl.BlockSpec((H2, H), lambda i: (0, 0)),
            pl.BlockSpec((1, H), lambda i: (0, 0)),
            pl.BlockSpec((H, H2), lambda i: (0, 0)),
            pl.BlockSpec((H, H2), lambda i: (0, 0)),
        ],
        out_specs=[
            pl.BlockSpec((BN, H), lambda i: (i, 0)),
            pl.BlockSpec((BN, H2), lambda i: (i, 0)),
            pl.BlockSpec((BN, H2), lambda i: (i, 0)),
        ],
        out_shape=[
            jax.ShapeDtypeStruct((N, H), jnp.float32),
            jax.ShapeDtypeStruct((N, H2), jnp.float32),
            jax.ShapeDtypeStruct((N, H2), jnp.float32),
        ],
    )(nodes, wv1, _row2(bv1), wv2, _row2(bv2), wfe_c, wfe_n)

    table = jnp.concatenate([vc, vn], axis=0)
    gidx = jnp.concatenate([index[:, 0], index[:, 1] + N]).astype(jnp.int32)
    gcn = _sc_gather(table, gidx)

    BE = 3200
    epre = pl.pallas_call(
        _k2_body,
        grid=(E // BE,),
        in_specs=[
            pl.BlockSpec((BE, H), lambda i: (i, 0)),
            pl.BlockSpec((H, H2), lambda i: (0, 0)),
            pl.BlockSpec((1, H2), lambda i: (0, 0)),
            pl.BlockSpec((H2, H), lambda i: (0, 0)),
            pl.BlockSpec((1, H), lambda i: (0, 0)),
            pl.BlockSpec((H, H2), lambda i: (0, 0)),
        ],
        out_specs=[pl.BlockSpec((BE, H2), lambda i: (i, 0))],
        out_shape=[jax.ShapeDtypeStruct((E, H2), jnp.float32)],
    )(edges, we1, _row2(be1), we2, _row2(be2), wfe_e)[0]

    gu1_3d = gu1[:G].reshape(G, 1, H2)
    eidx_3d = edge_index.astype(jnp.int32).reshape(G, 1, PPG)
    e_out, ep, epau = pl.pallas_call(
        _k3_body,
        grid=(G,),
        in_specs=[
            pl.BlockSpec((PPG, H2), lambda g: (g, 0)),
            pl.BlockSpec((PPG, H2), lambda g: (g, 0)),
            pl.BlockSpec((PPG, H2), lambda g, _G=G: (g + _G, 0)),
            pl.BlockSpec((PPG, H), lambda g: (g, 0)),
            pl.BlockSpec((1, 1, H2), lambda g: (g, 0, 0)),
            pl.BlockSpec((1, 1, PPG), lambda g: (g, 0, 0)),
            pl.BlockSpec((1, H2), lambda g: (0, 0)),
            pl.BlockSpec((H2, H2), lambda g: (0, 0)),
            pl.BlockSpec((1, H2), lambda g: (0, 0)),
            pl.BlockSpec((H2, H), lambda g: (0, 0)),
            pl.BlockSpec((1, H), lambda g: (0, 0)),
        ],
        out_specs=[
            pl.BlockSpec((PPG, H), lambda g: (g, 0)),
            pl.BlockSpec((PPG, H), lambda g: (g, 0)),
            pl.BlockSpec((GP, H), lambda g: (0, 0)),
        ],
        out_shape=[
            jax.ShapeDtypeStruct((E, H), jnp.float32),
            jax.ShapeDtypeStruct((E, H), jnp.float32),
            jax.ShapeDtypeStruct((GP, H), jnp.float32),
        ],
    )(epre, gcn, gcn, edges, gu1_3d, eidx_3d, _row2(bfe1), wfe2,
      _row2(bfe2), wfe3, _row2(bfe3))

    parts = _sc_scatter_add(ep, index[:, 0].astype(jnp.int32), N)

    nidx_3d = node_index.astype(jnp.int32).reshape(N // BN, 1, BN)
    v_out, vpau = pl.pallas_call(
        functools.partial(_k4_body, npg=NPG, gpb=BN // NPG),
        grid=(N // BN,),
        in_specs=[
            pl.BlockSpec((1, BN, H), lambda i: (0, i, 0)),
            pl.BlockSpec((1, BN, H), lambda i: (1, i, 0)),
            pl.BlockSpec((BN, H), lambda i: (i, 0)),
            pl.BlockSpec((BN, H), lambda i: (i, 0)),
            pl.BlockSpec((GP, H2), lambda i: (0, 0)),
            pl.BlockSpec((1, 1, BN), lambda i: (i, 0, 0)),
            pl.BlockSpec((H, H2), lambda i: (0, 0)),
            pl.BlockSpec((H, H2), lambda i: (0, 0)),
            pl.BlockSpec((1, H2), lambda i: (0, 0)),
            pl.BlockSpec((H2, H2), lambda i: (0, 0)),
            pl.BlockSpec((1, H2), lambda i: (0, 0)),
            pl.BlockSpec((H2, H), lambda i: (0, 0)),
            pl.BlockSpec((1, H), lambda i: (0, 0)),
        ],
        out_specs=[
            pl.BlockSpec((BN, H), lambda i: (i, 0)),
            pl.BlockSpec((GP, H), lambda i: (0, 0)),
        ],
        out_shape=[
            jax.ShapeDtypeStruct((N, H), jnp.float32),
            jax.ShapeDtypeStruct((GP, H), jnp.float32),
        ],
    )(parts, parts, v, nodes, gv1, nidx_3d, wfv_a, wfv_b, _row2(bfv1),
      wfv2, _row2(bfv2), wfv3, _row2(bfv3))

    u_out_p = pl.pallas_call(
        _k5_body,
        out_shape=[jax.ShapeDtypeStruct((GP, H), jnp.float32)],
    )(epau, vpau, u_p, state_p, wfu_a, wfu_b, wfu_c, _row2(bfu1), wfu2,
      _row2(bfu2), wfu3, _row2(bfu3))[0]

    return (e_out, v_out, u_out_p[:G])


# speedup vs baseline: 3.7114x; 1.1467x over previous
import functools

import jax
import jax.numpy as jnp
import numpy as np
from jax.experimental import pallas as pl
from jax.experimental.pallas import tpu as pltpu
from jax.experimental.pallas import tpu_sc as plsc

_LOG2 = float(np.log(2.0))

_SC_CORES = 2
_SC_SUBCORES = 16


def _ssp(x):
    return jnp.maximum(x, 0.0) + jnp.log1p(jnp.exp(-jnp.abs(x))) - _LOG2


def _sc_gather(table, idx):
    K = idx.shape[0]
    D = table.shape[1]
    W = 256
    mesh = plsc.VectorSubcoreMesh(core_axis_name="core", subcore_axis_name="subcore")

    @pl.kernel(out_type=jax.ShapeDtypeStruct((K, D), table.dtype), mesh=mesh)
    def k(t_hbm, i_hbm, o_hbm):
        def body(i_vmem, o_vmem):
            pltpu.sync_copy(t_hbm.at[i_vmem.at[0]], o_vmem)

        pltpu.emit_pipeline(
            body,
            grid=(K // W,),
            in_specs=[pl.BlockSpec((1, W), lambda i: (0, i))],
            out_specs=[pl.BlockSpec((W, D), lambda i: (i, 0))],
            core_axis_name=("core", "subcore"),
            dimension_semantics=(pltpu.PARALLEL,),
        )(i_hbm, o_hbm)

    return k(table, idx.reshape(1, K))


def _sc_scatter_add(ep, idx, n_rows):
    E, H = ep.shape
    CHUNK = 128
    n_pad = ((n_rows + 8 * _SC_SUBCORES - 1) // (8 * _SC_SUBCORES)) * (8 * _SC_SUBCORES)
    RPT = n_pad // _SC_SUBCORES
    mesh = plsc.VectorSubcoreMesh(core_axis_name="core", subcore_axis_name="subcore")
    zeros = jnp.zeros((RPT, H), dtype=ep.dtype)

    @pl.kernel(
        out_type=jax.ShapeDtypeStruct((_SC_CORES, n_pad, H), ep.dtype),
        mesh=mesh,
        scratch_types=[pltpu.VMEM_SHARED((n_pad, H), ep.dtype)],
    )
    def k(ep_hbm, i_hbm, z_hbm, o_hbm, acc):
        core = jax.lax.axis_index("core")
        sub = jax.lax.axis_index("subcore")
        pltpu.sync_copy(z_hbm, acc.at[pl.ds(sub * RPT, RPT)])
        plsc.subcore_barrier()

        def body(ep_vmem, i_vmem):
            pltpu.sync_copy(ep_vmem, acc.at[i_vmem.at[0]], add=True)

        pltpu.emit_pipeline(
            body,
            grid=(E // CHUNK,),
            in_specs=[
                pl.BlockSpec((CHUNK, H), lambda i: (i, 0)),
                pl.BlockSpec((1, CHUNK), lambda i: (0, i)),
            ],
            out_specs=[],
            core_axis_name=("core", "subcore"),
            dimension_semantics=(pltpu.PARALLEL,),
        )(ep_hbm, i_hbm)
        plsc.subcore_barrier()
        pltpu.sync_copy(
            acc.at[pl.ds(sub * RPT, RPT)],
            o_hbm.at[core].at[pl.ds(sub * RPT, RPT)],
        )

    return k(ep, idx.reshape(1, E), zeros)


def _k0_body(state_ref, wu1_ref, bu1_ref, wu2_ref, bu2_ref, wfeu_ref, wfvu_ref,
             u_ref, gu1_ref, gv1_ref):
    t = _ssp(state_ref[...] @ wu1_ref[...] + bu1_ref[...])
    u = _ssp(t @ wu2_ref[...] + bu2_ref[...])
    u_ref[...] = u
    gu1_ref[...] = u @ wfeu_ref[...]
    gv1_ref[...] = u @ wfvu_ref[...]


def _k1_body(nodes_ref, wv1_ref, bv1_ref, wv2_ref, bv2_ref, v_ref):
    t = _ssp(nodes_ref[...] @ wv1_ref[...] + bv1_ref[...])
    v_ref[...] = _ssp(t @ wv2_ref[...] + bv2_ref[...])


def _k2_body(edges_ref, we1_ref, be1_ref, we2_ref, be2_ref, wfee_ref,
             epre_ref):
    t = _ssp(edges_ref[...] @ we1_ref[...] + be1_ref[...])
    e = _ssp(t @ we2_ref[...] + be2_ref[...])
    epre_ref[...] = e @ wfee_ref[...]


def _k3_body(epre_ref, gc_ref, gn_ref, edges_ref, gu1_ref, eidx_ref,
             wfec_ref, wfen_ref,
             bfe1_ref, wfe2_ref, bfe2_ref, wfe3_ref, bfe3_ref,
             eout_ref, ep_ref, acc_ref):
    g = pl.program_id(0)
    B = epre_ref.shape[0]
    h1 = _ssp(epre_ref[...] + jax.lax.dot(gc_ref[...], wfec_ref[...])
              + jax.lax.dot(gn_ref[...], wfen_ref[...]) + gu1_ref[0]
              + bfe1_ref[...])
    h2 = _ssp(h1 @ wfe2_ref[...] + bfe2_ref[...])
    ep = _ssp(h2 @ wfe3_ref[...] + bfe3_ref[...])
    ep_ref[...] = ep
    eout_ref[...] = edges_ref[...] + ep
    eidx = eidx_ref[0]
    oh = (jax.lax.broadcasted_iota(jnp.int32, (128, B), 0) == eidx
          ).astype(jnp.float32)
    contrib = jax.lax.dot(oh, ep)

    @pl.when(g == 0)
    def _():
        acc_ref[...] = contrib

    @pl.when(g > 0)
    def _():
        acc_ref[...] += contrib


def _k4_body(p0_ref, p1_ref, v_ref, nodes_ref, gv1_ref, nidx_ref,
             wfva_ref, wfvb_ref, bfv1_ref, wfv2_ref, bfv2_ref, wfv3_ref,
             bfv3_ref, vout_ref, acc_ref, *, npg, gpb):
    g = pl.program_id(0)
    B = v_ref.shape[0]
    row_graph = g * gpb + jax.lax.broadcasted_iota(jnp.int32, (B, 128), 0) // npg
    ohu = (row_graph == jax.lax.broadcasted_iota(jnp.int32, (B, 128), 1)
           ).astype(jnp.float32)
    uv = jax.lax.dot(ohu, gv1_ref[...])
    epav = p0_ref[0] + p1_ref[0]
    h1 = _ssp(jax.lax.dot(epav, wfva_ref[...]) + jax.lax.dot(v_ref[...], wfvb_ref[...])
              + uv + bfv1_ref[...])
    h2 = _ssp(h1 @ wfv2_ref[...] + bfv2_ref[...])
    vp = _ssp(h2 @ wfv3_ref[...] + bfv3_ref[...])
    vout_ref[...] = nodes_ref[...] + vp
    nidx = nidx_ref[0]
    oh = (jax.lax.broadcasted_iota(jnp.int32, (128, B), 0) == nidx
          ).astype(jnp.float32)
    contrib = jax.lax.dot(oh, vp)

    @pl.when(g == 0)
    def _():
        acc_ref[...] = contrib

    @pl.when(g > 0)
    def _():
        acc_ref[...] += contrib


def _k5_body(epau_ref, vpau_ref, u_ref, state_ref,
             wfua_ref, wfub_ref, wfuc_ref, bfu1_ref, wfu2_ref, bfu2_ref,
             wfu3_ref, bfu3_ref, uout_ref):
    h1 = _ssp(jax.lax.dot(epau_ref[...], wfua_ref[...])
              + jax.lax.dot(vpau_ref[...], wfub_ref[...])
              + jax.lax.dot(u_ref[...], wfuc_ref[...]) + bfu1_ref[...])
    h2 = _ssp(h1 @ wfu2_ref[...] + bfu2_ref[...])
    up = _ssp(h2 @ wfu3_ref[...] + bfu3_ref[...])
    uout_ref[...] = state_ref[...] + up


def _row2(b):
    return b.reshape(1, -1)


def kernel(nodes, num_atoms, node_index, state, max_num_nbrs, num_pairs,
           edge_index, index, edges, params):
    N, H = nodes.shape
    E = edges.shape[0]
    G = state.shape[0]
    NPG = N // G
    PPG = E // G
    H2 = 2 * H

    (wv1, bv1), (wv2, bv2) = params["pv"]
    (we1, be1), (we2, be2) = params["pe"]
    (wu1, bu1), (wu2, bu2) = params["pu"]
    (wfe1, bfe1), (wfe2, bfe2), (wfe3, bfe3) = params["fe"]
    (wfv1, bfv1), (wfv2, bfv2), (wfv3, bfv3) = params["fv"]
    (wfu1, bfu1), (wfu2, bfu2), (wfu3, bfu3) = params["fu"]

    wfe_c, wfe_e, wfe_n, wfe_u = (wfe1[0:H], wfe1[H:2 * H], wfe1[2 * H:3 * H],
                                  wfe1[3 * H:4 * H])
    wfv_a, wfv_b, wfv_u = wfv1[0:H], wfv1[H:2 * H], wfv1[2 * H:3 * H]
    wfu_a, wfu_b, wfu_c = wfu1[0:H], wfu1[H:2 * H], wfu1[2 * H:3 * H]

    GP = 128
    state_p = jnp.zeros((GP, H), jnp.float32).at[:G].set(state)

    u_p, gu1, gv1 = pl.pallas_call(
        _k0_body,
        out_shape=[
            jax.ShapeDtypeStruct((GP, H), jnp.float32),
            jax.ShapeDtypeStruct((GP, H2), jnp.float32),
            jax.ShapeDtypeStruct((GP, H2), jnp.float32),
        ],
    )(state_p, wu1, _row2(bu1), wu2, _row2(bu2), wfe_u, wfv_u)

    BN = 1000
    v = pl.pallas_call(
        _k1_body,
        grid=(N // BN,),
        in_specs=[
            pl.BlockSpec((BN, H), lambda i: (i, 0)),
            pl.BlockSpec((H, H2), lambda i: (0, 0)),
            pl.BlockSpec((1, H2), lambda i: (0, 0)),
            pl.BlockSpec((H2, H), lambda i: (0, 0)),
            pl.BlockSpec((1, H), lambda i: (0, 0)),
        ],
        out_specs=[pl.BlockSpec((BN, H), lambda i: (i, 0))],
        out_shape=[jax.ShapeDtypeStruct((N, H), jnp.float32)],
    )(nodes, wv1, _row2(bv1), wv2, _row2(bv2))[0]

    gidx = jnp.concatenate([index[:, 0], index[:, 1]]).astype(jnp.int32)
    gcn = _sc_gather(v, gidx)

    BE = 3200
    epre = pl.pallas_call(
        _k2_body,
        grid=(E // BE,),
        in_specs=[
            pl.BlockSpec((BE, H), lambda i: (i, 0)),
            pl.BlockSpec((H, H2), lambda i: (0, 0)),
            pl.BlockSpec((1, H2), lambda i: (0, 0)),
            pl.BlockSpec((H2, H), lambda i: (0, 0)),
            pl.BlockSpec((1, H), lambda i: (0, 0)),
            pl.BlockSpec((H, H2), lambda i: (0, 0)),
        ],
        out_specs=[pl.BlockSpec((BE, H2), lambda i: (i, 0))],
        out_shape=[jax.ShapeDtypeStruct((E, H2), jnp.float32)],
    )(edges, we1, _row2(be1), we2, _row2(be2), wfe_e)[0]

    gu1_3d = gu1[:G].reshape(G, 1, H2)
    eidx_3d = edge_index.astype(jnp.int32).reshape(G, 1, PPG)
    e_out, ep, epau = pl.pallas_call(
        _k3_body,
        grid=(G,),
        in_specs=[
            pl.BlockSpec((PPG, H2), lambda g: (g, 0)),
            pl.BlockSpec((PPG, H), lambda g: (g, 0)),
            pl.BlockSpec((PPG, H), lambda g, _G=G: (g + _G, 0)),
            pl.BlockSpec((PPG, H), lambda g: (g, 0)),
            pl.BlockSpec((1, 1, H2), lambda g: (g, 0, 0)),
            pl.BlockSpec((1, 1, PPG), lambda g: (g, 0, 0)),
            pl.BlockSpec((H, H2), lambda g: (0, 0)),
            pl.BlockSpec((H, H2), lambda g: (0, 0)),
            pl.BlockSpec((1, H2), lambda g: (0, 0)),
            pl.BlockSpec((H2, H2), lambda g: (0, 0)),
            pl.BlockSpec((1, H2), lambda g: (0, 0)),
            pl.BlockSpec((H2, H), lambda g: (0, 0)),
            pl.BlockSpec((1, H), lambda g: (0, 0)),
        ],
        out_specs=[
            pl.BlockSpec((PPG, H), lambda g: (g, 0)),
            pl.BlockSpec((PPG, H), lambda g: (g, 0)),
            pl.BlockSpec((GP, H), lambda g: (0, 0)),
        ],
        out_shape=[
            jax.ShapeDtypeStruct((E, H), jnp.float32),
            jax.ShapeDtypeStruct((E, H), jnp.float32),
            jax.ShapeDtypeStruct((GP, H), jnp.float32),
        ],
    )(epre, gcn, gcn, edges, gu1_3d, eidx_3d, wfe_c, wfe_n, _row2(bfe1),
      wfe2, _row2(bfe2), wfe3, _row2(bfe3))

    parts = _sc_scatter_add(ep, index[:, 0].astype(jnp.int32), N)

    nidx_3d = node_index.astype(jnp.int32).reshape(N // BN, 1, BN)
    v_out, vpau = pl.pallas_call(
        functools.partial(_k4_body, npg=NPG, gpb=BN // NPG),
        grid=(N // BN,),
        in_specs=[
            pl.BlockSpec((1, BN, H), lambda i: (0, i, 0)),
            pl.BlockSpec((1, BN, H), lambda i: (1, i, 0)),
            pl.BlockSpec((BN, H), lambda i: (i, 0)),
            pl.BlockSpec((BN, H), lambda i: (i, 0)),
            pl.BlockSpec((GP, H2), lambda i: (0, 0)),
            pl.BlockSpec((1, 1, BN), lambda i: (i, 0, 0)),
            pl.BlockSpec((H, H2), lambda i: (0, 0)),
            pl.BlockSpec((H, H2), lambda i: (0, 0)),
            pl.BlockSpec((1, H2), lambda i: (0, 0)),
            pl.BlockSpec((H2, H2), lambda i: (0, 0)),
            pl.BlockSpec((1, H2), lambda i: (0, 0)),
            pl.BlockSpec((H2, H), lambda i: (0, 0)),
            pl.BlockSpec((1, H), lambda i: (0, 0)),
        ],
        out_specs=[
            pl.BlockSpec((BN, H), lambda i: (i, 0)),
            pl.BlockSpec((GP, H), lambda i: (0, 0)),
        ],
        out_shape=[
            jax.ShapeDtypeStruct((N, H), jnp.float32),
            jax.ShapeDtypeStruct((GP, H), jnp.float32),
        ],
    )(parts, parts, v, nodes, gv1, nidx_3d, wfv_a, wfv_b, _row2(bfv1),
      wfv2, _row2(bfv2), wfv3, _row2(bfv3))

    u_out_p = pl.pallas_call(
        _k5_body,
        out_shape=[jax.ShapeDtypeStruct((GP, H), jnp.float32)],
    )(epau, vpau, u_p, state_p, wfu_a, wfu_b, wfu_c, _row2(bfu1), wfu2,
      _row2(bfu2), wfu3, _row2(bfu3))[0]

    return (e_out, v_out, u_out_p[:G])


# speedup vs baseline: 4.0974x; 1.1040x over previous
import functools

import jax
import jax.numpy as jnp
import numpy as np
from jax.experimental import pallas as pl
from jax.experimental.pallas import tpu as pltpu
from jax.experimental.pallas import tpu_sc as plsc

_LOG2 = float(np.log(2.0))

_SC_CORES = 2
_SC_SUBCORES = 16


_LOG2E = 1.4426950408889634


def _ssp(x):
    u = jnp.exp2(jnp.abs(x) * (-_LOG2E))
    return jnp.maximum(x, 0.0) + _LOG2 * (jnp.log2(1.0 + u) - 1.0)


def _sc_gather(table, idx):
    K = idx.shape[0]
    D = table.shape[1]
    W = 256
    mesh = plsc.VectorSubcoreMesh(core_axis_name="core", subcore_axis_name="subcore")

    @pl.kernel(out_type=jax.ShapeDtypeStruct((K, D), table.dtype), mesh=mesh)
    def k(t_hbm, i_hbm, o_hbm):
        def body(i_vmem, o_vmem):
            pltpu.sync_copy(t_hbm.at[i_vmem.at[0]], o_vmem)

        pltpu.emit_pipeline(
            body,
            grid=(K // W,),
            in_specs=[pl.BlockSpec((1, W), lambda i: (0, i))],
            out_specs=[pl.BlockSpec((W, D), lambda i: (i, 0))],
            core_axis_name=("core", "subcore"),
            dimension_semantics=(pltpu.PARALLEL,),
        )(i_hbm, o_hbm)

    return k(table, idx.reshape(1, K))


def _sc_scatter_add(ep, idx, n_rows):
    E, H = ep.shape
    CHUNK = 128
    n_pad = ((n_rows + 8 * _SC_SUBCORES - 1) // (8 * _SC_SUBCORES)) * (8 * _SC_SUBCORES)
    RPT = n_pad // _SC_SUBCORES
    mesh = plsc.VectorSubcoreMesh(core_axis_name="core", subcore_axis_name="subcore")
    zeros = jnp.zeros((RPT, H), dtype=ep.dtype)

    @pl.kernel(
        out_type=jax.ShapeDtypeStruct((_SC_CORES, n_pad, H), ep.dtype),
        mesh=mesh,
        scratch_types=[pltpu.VMEM_SHARED((n_pad, H), ep.dtype)],
    )
    def k(ep_hbm, i_hbm, z_hbm, o_hbm, acc):
        core = jax.lax.axis_index("core")
        sub = jax.lax.axis_index("subcore")
        pltpu.sync_copy(z_hbm, acc.at[pl.ds(sub * RPT, RPT)])
        plsc.subcore_barrier()

        def body(ep_vmem, i_vmem):
            pltpu.sync_copy(ep_vmem, acc.at[i_vmem.at[0]], add=True)

        pltpu.emit_pipeline(
            body,
            grid=(E // CHUNK,),
            in_specs=[
                pl.BlockSpec((CHUNK, H), lambda i: (i, 0)),
                pl.BlockSpec((1, CHUNK), lambda i: (0, i)),
            ],
            out_specs=[],
            core_axis_name=("core", "subcore"),
            dimension_semantics=(pltpu.PARALLEL,),
        )(ep_hbm, i_hbm)
        plsc.subcore_barrier()
        pltpu.sync_copy(
            acc.at[pl.ds(sub * RPT, RPT)],
            o_hbm.at[core].at[pl.ds(sub * RPT, RPT)],
        )

    return k(ep, idx.reshape(1, E), zeros)


def _k0_body(state_ref, wu1_ref, bu1_ref, wu2_ref, bu2_ref, wfeu_ref, wfvu_ref,
             u_ref, gu1_ref, gv1_ref):
    t = _ssp(state_ref[...] @ wu1_ref[...] + bu1_ref[...])
    u = _ssp(t @ wu2_ref[...] + bu2_ref[...])
    u_ref[...] = u
    gu1_ref[...] = u @ wfeu_ref[...]
    gv1_ref[...] = u @ wfvu_ref[...]


def _k1_body(nodes_ref, wv1_ref, bv1_ref, wv2_ref, bv2_ref, v_ref):
    t = _ssp(nodes_ref[...] @ wv1_ref[...] + bv1_ref[...])
    v_ref[...] = _ssp(t @ wv2_ref[...] + bv2_ref[...])


def _k2_body(edges_ref, we1_ref, be1_ref, we2_ref, be2_ref, wfee_ref,
             epre_ref):
    t = _ssp(edges_ref[...] @ we1_ref[...] + be1_ref[...])
    e = _ssp(t @ we2_ref[...] + be2_ref[...])
    epre_ref[...] = e @ wfee_ref[...]


def _k3_body(epre_ref, gc_ref, gn_ref, edges_ref, gu1_ref, eidx_ref,
             wfec_ref, wfen_ref,
             bfe1_ref, wfe2_ref, bfe2_ref, wfe3_ref, bfe3_ref,
             eout_ref, ep_ref, acc_ref):
    g = pl.program_id(0)
    B = epre_ref.shape[0]
    h1 = _ssp(epre_ref[...] + jax.lax.dot(gc_ref[...], wfec_ref[...])
              + jax.lax.dot(gn_ref[...], wfen_ref[...]) + gu1_ref[0]
              + bfe1_ref[...])
    h2 = _ssp(h1 @ wfe2_ref[...] + bfe2_ref[...])
    ep = _ssp(h2 @ wfe3_ref[...] + bfe3_ref[...])
    ep_ref[...] = ep
    eout_ref[...] = edges_ref[...] + ep
    eidx = eidx_ref[0]
    oh = (jax.lax.broadcasted_iota(jnp.int32, (128, B), 0) == eidx
          ).astype(jnp.float32)
    contrib = jax.lax.dot(oh, ep)

    @pl.when(g == 0)
    def _():
        acc_ref[...] = contrib

    @pl.when(g > 0)
    def _():
        acc_ref[...] += contrib


def _k4_body(p0_ref, p1_ref, v_ref, nodes_ref, gv1_ref, nidx_ref,
             wfva_ref, wfvb_ref, bfv1_ref, wfv2_ref, bfv2_ref, wfv3_ref,
             bfv3_ref, vout_ref, acc_ref, *, npg, gpb):
    g = pl.program_id(0)
    B = v_ref.shape[0]
    row_graph = g * gpb + jax.lax.broadcasted_iota(jnp.int32, (B, 128), 0) // npg
    ohu = (row_graph == jax.lax.broadcasted_iota(jnp.int32, (B, 128), 1)
           ).astype(jnp.float32)
    uv = jax.lax.dot(ohu, gv1_ref[...])
    epav = p0_ref[0] + p1_ref[0]
    h1 = _ssp(jax.lax.dot(epav, wfva_ref[...]) + jax.lax.dot(v_ref[...], wfvb_ref[...])
              + uv + bfv1_ref[...])
    h2 = _ssp(h1 @ wfv2_ref[...] + bfv2_ref[...])
    vp = _ssp(h2 @ wfv3_ref[...] + bfv3_ref[...])
    vout_ref[...] = nodes_ref[...] + vp
    nidx = nidx_ref[0]
    oh = (jax.lax.broadcasted_iota(jnp.int32, (128, B), 0) == nidx
          ).astype(jnp.float32)
    contrib = jax.lax.dot(oh, vp)

    @pl.when(g == 0)
    def _():
        acc_ref[...] = contrib

    @pl.when(g > 0)
    def _():
        acc_ref[...] += contrib


def _k5_body(epau_ref, vpau_ref, u_ref, state_ref,
             wfua_ref, wfub_ref, wfuc_ref, bfu1_ref, wfu2_ref, bfu2_ref,
             wfu3_ref, bfu3_ref, uout_ref):
    h1 = _ssp(jax.lax.dot(epau_ref[...], wfua_ref[...])
              + jax.lax.dot(vpau_ref[...], wfub_ref[...])
              + jax.lax.dot(u_ref[...], wfuc_ref[...]) + bfu1_ref[...])
    h2 = _ssp(h1 @ wfu2_ref[...] + bfu2_ref[...])
    up = _ssp(h2 @ wfu3_ref[...] + bfu3_ref[...])
    uout_ref[...] = state_ref[...] + up


def _row2(b):
    return b.reshape(1, -1)


def kernel(nodes, num_atoms, node_index, state, max_num_nbrs, num_pairs,
           edge_index, index, edges, params):
    N, H = nodes.shape
    E = edges.shape[0]
    G = state.shape[0]
    NPG = N // G
    PPG = E // G
    H2 = 2 * H

    (wv1, bv1), (wv2, bv2) = params["pv"]
    (we1, be1), (we2, be2) = params["pe"]
    (wu1, bu1), (wu2, bu2) = params["pu"]
    (wfe1, bfe1), (wfe2, bfe2), (wfe3, bfe3) = params["fe"]
    (wfv1, bfv1), (wfv2, bfv2), (wfv3, bfv3) = params["fv"]
    (wfu1, bfu1), (wfu2, bfu2), (wfu3, bfu3) = params["fu"]

    wfe_c, wfe_e, wfe_n, wfe_u = (wfe1[0:H], wfe1[H:2 * H], wfe1[2 * H:3 * H],
                                  wfe1[3 * H:4 * H])
    wfv_a, wfv_b, wfv_u = wfv1[0:H], wfv1[H:2 * H], wfv1[2 * H:3 * H]
    wfu_a, wfu_b, wfu_c = wfu1[0:H], wfu1[H:2 * H], wfu1[2 * H:3 * H]

    GP = 128
    state_p = jnp.zeros((GP, H), jnp.float32).at[:G].set(state)

    u_p, gu1, gv1 = pl.pallas_call(
        _k0_body,
        out_shape=[
            jax.ShapeDtypeStruct((GP, H), jnp.float32),
            jax.ShapeDtypeStruct((GP, H2), jnp.float32),
            jax.ShapeDtypeStruct((GP, H2), jnp.float32),
        ],
    )(state_p, wu1, _row2(bu1), wu2, _row2(bu2), wfe_u, wfv_u)

    BN = 1000
    v = pl.pallas_call(
        _k1_body,
        grid=(N // BN,),
        in_specs=[
            pl.BlockSpec((BN, H), lambda i: (i, 0)),
            pl.BlockSpec((H, H2), lambda i: (0, 0)),
            pl.BlockSpec((1, H2), lambda i: (0, 0)),
            pl.BlockSpec((H2, H), lambda i: (0, 0)),
            pl.BlockSpec((1, H), lambda i: (0, 0)),
        ],
        out_specs=[pl.BlockSpec((BN, H), lambda i: (i, 0))],
        out_shape=[jax.ShapeDtypeStruct((N, H), jnp.float32)],
    )(nodes, wv1, _row2(bv1), wv2, _row2(bv2))[0]

    BE = 3200
    epre = pl.pallas_call(
        _k2_body,
        grid=(E // BE,),
        in_specs=[
            pl.BlockSpec((BE, H), lambda i: (i, 0)),
            pl.BlockSpec((H, H2), lambda i: (0, 0)),
            pl.BlockSpec((1, H2), lambda i: (0, 0)),
            pl.BlockSpec((H2, H), lambda i: (0, 0)),
            pl.BlockSpec((1, H), lambda i: (0, 0)),
            pl.BlockSpec((H, H2), lambda i: (0, 0)),
        ],
        out_specs=[pl.BlockSpec((BE, H2), lambda i: (i, 0))],
        out_shape=[jax.ShapeDtypeStruct((E, H2), jnp.float32)],
    )(edges, we1, _row2(be1), we2, _row2(be2), wfe_e)[0]

    gidx = jnp.concatenate([index[:, 0], index[:, 1]]).astype(jnp.int32)
    gcn = _sc_gather(v, gidx)

    gu1_3d = gu1[:G].reshape(G, 1, H2)
    eidx_3d = edge_index.astype(jnp.int32).reshape(G, 1, PPG)
    e_out, ep, epau = pl.pallas_call(
        _k3_body,
        grid=(G,),
        in_specs=[
            pl.BlockSpec((PPG, H2), lambda g: (g, 0)),
            pl.BlockSpec((PPG, H), lambda g: (g, 0)),
            pl.BlockSpec((PPG, H), lambda g, _G=G: (g + _G, 0)),
            pl.BlockSpec((PPG, H), lambda g: (g, 0)),
            pl.BlockSpec((1, 1, H2), lambda g: (g, 0, 0)),
            pl.BlockSpec((1, 1, PPG), lambda g: (g, 0, 0)),
            pl.BlockSpec((H, H2), lambda g: (0, 0)),
            pl.BlockSpec((H, H2), lambda g: (0, 0)),
            pl.BlockSpec((1, H2), lambda g: (0, 0)),
            pl.BlockSpec((H2, H2), lambda g: (0, 0)),
            pl.BlockSpec((1, H2), lambda g: (0, 0)),
            pl.BlockSpec((H2, H), lambda g: (0, 0)),
            pl.BlockSpec((1, H), lambda g: (0, 0)),
        ],
        out_specs=[
            pl.BlockSpec((PPG, H), lambda g: (g, 0)),
            pl.BlockSpec((PPG, H), lambda g: (g, 0)),
            pl.BlockSpec((GP, H), lambda g: (0, 0)),
        ],
        out_shape=[
            jax.ShapeDtypeStruct((E, H), jnp.float32),
            jax.ShapeDtypeStruct((E, H), jnp.float32),
            jax.ShapeDtypeStruct((GP, H), jnp.float32),
        ],
    )(epre, gcn, gcn, edges, gu1_3d, eidx_3d, wfe_c, wfe_n, _row2(bfe1),
      wfe2, _row2(bfe2), wfe3, _row2(bfe3))

    parts = _sc_scatter_add(ep, index[:, 0].astype(jnp.int32), N)

    nidx_3d = node_index.astype(jnp.int32).reshape(N // BN, 1, BN)
    v_out, vpau = pl.pallas_call(
        functools.partial(_k4_body, npg=NPG, gpb=BN // NPG),
        grid=(N // BN,),
        in_specs=[
            pl.BlockSpec((1, BN, H), lambda i: (0, i, 0)),
            pl.BlockSpec((1, BN, H), lambda i: (1, i, 0)),
            pl.BlockSpec((BN, H), lambda i: (i, 0)),
            pl.BlockSpec((BN, H), lambda i: (i, 0)),
            pl.BlockSpec((GP, H2), lambda i: (0, 0)),
            pl.BlockSpec((1, 1, BN), lambda i: (i, 0, 0)),
            pl.BlockSpec((H, H2), lambda i: (0, 0)),
            pl.BlockSpec((H, H2), lambda i: (0, 0)),
            pl.BlockSpec((1, H2), lambda i: (0, 0)),
            pl.BlockSpec((H2, H2), lambda i: (0, 0)),
            pl.BlockSpec((1, H2), lambda i: (0, 0)),
            pl.BlockSpec((H2, H), lambda i: (0, 0)),
            pl.BlockSpec((1, H), lambda i: (0, 0)),
        ],
        out_specs=[
            pl.BlockSpec((BN, H), lambda i: (i, 0)),
            pl.BlockSpec((GP, H), lambda i: (0, 0)),
        ],
        out_shape=[
            jax.ShapeDtypeStruct((N, H), jnp.float32),
            jax.ShapeDtypeStruct((GP, H), jnp.float32),
        ],
    )(parts, parts, v, nodes, gv1, nidx_3d, wfv_a, wfv_b, _row2(bfv1),
      wfv2, _row2(bfv2), wfv3, _row2(bfv3))

    u_out_p = pl.pallas_call(
        _k5_body,
        out_shape=[jax.ShapeDtypeStruct((GP, H), jnp.float32)],
    )(epau, vpau, u_p, state_p, wfu_a, wfu_b, wfu_c, _row2(bfu1), wfu2,
      _row2(bfu2), wfu3, _row2(bfu3))[0]

    return (e_out, v_out, u_out_p[:G])
